# baseline (speedup 1.0000x reference)
import functools

import numpy as np
import jax
import jax.numpy as jnp
from jax import lax
from jax.experimental import pallas as pl
from jax.experimental.pallas import tpu as pltpu
from jax.experimental.pallas import tpu_sc as plsc

_N, _D, _H, _E, _K = 2048, 768, 3072, 8, 2
_BLOCK = 128
_NBLK = (_N * _K) // _BLOCK + _E
_PAD = _NBLK * _BLOCK
_NPAIR = _N * _K
_EPS = float(np.finfo(np.float64).eps)
_NW = 32


def _gate_body(x_ref, wg_ref, p_ref, g_ref, ri_ref, first_ref, rex_ref,
               nr_ref, bu_ref):
    x = x_ref[...]
    wg = wg_ref[...]
    logits = lax.dot_general(x, wg, (((1,), (1,)), ((), ())),
                             preferred_element_type=jnp.float32)
    ioe = lax.broadcasted_iota(jnp.int32, (_N, _E), 1)
    m1 = jnp.max(logits, axis=1, keepdims=True)
    i1 = jnp.min(jnp.where(logits == m1, ioe, _E), axis=1, keepdims=True)
    l2 = jnp.where(ioe == i1, jnp.float32(-jnp.inf), logits)
    m2 = jnp.max(l2, axis=1, keepdims=True)
    i2 = jnp.min(jnp.where(l2 == m2, ioe, _E), axis=1, keepdims=True)
    e21 = jnp.exp(m2 - m1)
    g1 = 1.0 / (1.0 + e21)
    g2 = e21 / (1.0 + e21)

    oh1 = (ioe == i1).astype(jnp.float32)
    oh2 = (ioe == i2).astype(jnp.float32)
    m_oh = jnp.concatenate([oh1, oh2], axis=0)

    bs = 512
    ti = lax.broadcasted_iota(jnp.int32, (bs, bs), 0)
    tj = lax.broadcasted_iota(jnp.int32, (bs, bs), 1)
    tri = (ti > tj).astype(jnp.float32)
    run = jnp.zeros((1, _E), jnp.float32)
    rank_rows = []
    for j in range(_NPAIR // bs):
        mj = m_oh[j * bs:(j + 1) * bs]
        rank_rows.append(
            lax.dot_general(tri, mj, (((1,), (0,)), ((), ())),
                            preferred_element_type=jnp.float32) + run)
        run = run + jnp.sum(mj, axis=0, keepdims=True)
    ranks = jnp.concatenate(rank_rows, axis=0)
    counts = run

    nb = jnp.floor((counts + (_BLOCK - 1)) / _BLOCK)
    si = lax.broadcasted_iota(jnp.int32, (_E, _E), 0)
    sj = lax.broadcasted_iota(jnp.int32, (_E, _E), 1)
    sl = (si < sj).astype(jnp.float32)
    bo = lax.dot_general(nb, sl, (((1,), (0,)), ((), ())),
                         preferred_element_type=jnp.float32)
    cnb = bo + nb

    slot_base = lax.dot_general(m_oh, bo * _BLOCK, (((1,), (1,)), ((), ())),
                                preferred_element_type=jnp.float32)
    rank_r = jnp.sum(ranks * m_oh, axis=1, keepdims=True)
    p_ref[...] = (slot_base + rank_r).astype(jnp.int32)
    g_ref[...] = jnp.concatenate([g1, g2], axis=0)

    iobi = lax.broadcasted_iota(jnp.int32, (_NBLK, _E), 0)
    nbpos = nb > 0
    first_m = jnp.logical_and(iobi == bo.astype(jnp.int32),
                              jnp.broadcast_to(nbpos, (_NBLK, _E)))
    first = jnp.sum(first_m.astype(jnp.float32), axis=1, keepdims=True)
    t24i = lax.broadcasted_iota(jnp.int32, (_NBLK, _NBLK), 0)
    t24j = lax.broadcasted_iota(jnp.int32, (_NBLK, _NBLK), 1)
    t24 = (t24i >= t24j).astype(jnp.float32)
    ri = lax.dot_general(t24, first, (((1,), (0,)), ((), ())),
                         preferred_element_type=jnp.float32) - 1.0
    ridx = lax.dot_general(nbpos.astype(jnp.float32), sl,
                           (((1,), (0,)), ((), ())),
                           preferred_element_type=jnp.float32)
    iorr = lax.broadcasted_iota(jnp.int32, (_E, _E), 0)
    run_m = jnp.logical_and(iorr == ridx.astype(jnp.int32),
                            jnp.broadcast_to(nbpos, (_E, _E)))
    lanecol = lax.broadcasted_iota(jnp.int32, (_E, 1), 0).astype(jnp.float32)
    rex = lax.dot_general(run_m.astype(jnp.float32), lanecol,
                          (((1,), (0,)), ((), ())),
                          preferred_element_type=jnp.float32)
    nr = jnp.sum(nbpos.astype(jnp.float32), axis=1, keepdims=True)
    ri_ref[...] = ri.astype(jnp.int32)
    first_ref[...] = first.astype(jnp.int32)
    rex_ref[...] = rex.astype(jnp.int32)
    nr_ref[...] = nr.astype(jnp.int32)
    total = jnp.sum(nb, axis=1, keepdims=True)
    iob1 = lax.broadcasted_iota(jnp.int32, (_NBLK, 1), 0).astype(jnp.float32)
    bu_ref[...] = (iob1 < total).astype(jnp.int32)


def _gating(x, w_gate):
    return pl.pallas_call(
        _gate_body,
        out_shape=(
            jax.ShapeDtypeStruct((_NPAIR, 1), jnp.int32),
            jax.ShapeDtypeStruct((_NPAIR, 1), jnp.float32),
            jax.ShapeDtypeStruct((_NBLK, 1), jnp.int32),
            jax.ShapeDtypeStruct((_NBLK, 1), jnp.int32),
            jax.ShapeDtypeStruct((_E, 1), jnp.int32),
            jax.ShapeDtypeStruct((1, 1), jnp.int32),
            jax.ShapeDtypeStruct((_NBLK, 1), jnp.int32),
        ),
    )(x, w_gate)


def _dispatch(x, p):
    ch = _NPAIR // _NW
    hc = ch // 2
    mesh = plsc.VectorSubcoreMesh(core_axis_name="c", subcore_axis_name="s")

    @functools.partial(
        pl.kernel, mesh=mesh,
        out_type=jax.ShapeDtypeStruct((_PAD, _D), jnp.float32),
        scratch_types=[
            pltpu.VMEM((2, hc), jnp.int32),
            pltpu.VMEM((2, hc, _D), jnp.float32),
            pltpu.SemaphoreType.DMA((2,)),
            pltpu.SemaphoreType.DMA((2,)),
        ],
    )
    def k(x_hbm, p_hbm, xs_hbm, idx_v, rows_v, lsem, ssem):
        wid = lax.axis_index("s") * 2 + lax.axis_index("c")
        base = wid * ch
        xb = lax.rem(base, _N)
        pltpu.sync_copy(p_hbm.at[pl.ds(base, hc)], idx_v.at[0])
        pltpu.sync_copy(p_hbm.at[pl.ds(base + hc, hc)], idx_v.at[1])
        l0 = pltpu.make_async_copy(x_hbm.at[pl.ds(xb, hc)],
                                   rows_v.at[0], lsem.at[0])
        l1 = pltpu.make_async_copy(x_hbm.at[pl.ds(xb + hc, hc)],
                                   rows_v.at[1], lsem.at[1])
        l0.start()
        l1.start()
        l0.wait()
        s0 = pltpu.make_async_copy(rows_v.at[0], xs_hbm.at[idx_v.at[0]],
                                   ssem.at[0])
        s0.start()
        l1.wait()
        s1 = pltpu.make_async_copy(rows_v.at[1], xs_hbm.at[idx_v.at[1]],
                                   ssem.at[1])
        s1.start()
        s0.wait()
        s1.wait()

    return k(x, p)


def _ffn_body(ri_ref, first_ref, rex_ref, nr_ref, bu_ref,
              xs_ref, w1_hbm, b1_ref, w2_hbm, b2_ref, out_ref,
              w1_buf, w2_buf, sem1, sem2):
    b = pl.program_id(0)
    ri = ri_ref[b]
    slot = lax.rem(ri, 2)

    def _start_fetch(run, slot_):
        e = rex_ref[run]
        pltpu.make_async_copy(w1_hbm.at[e], w1_buf.at[slot_],
                              sem1.at[slot_]).start()
        pltpu.make_async_copy(w2_hbm.at[e], w2_buf.at[slot_],
                              sem2.at[slot_]).start()

    @pl.when(b == 0)
    def _():
        _start_fetch(0, 0)

    @pl.when(first_ref[b] == 1)
    def _():
        nxt = ri + 1

        @pl.when(nxt < nr_ref[0])
        def _():
            _start_fetch(nxt, lax.rem(nxt, 2))

        e = rex_ref[ri]
        pltpu.make_async_copy(w1_hbm.at[e], w1_buf.at[slot],
                              sem1.at[slot]).wait()
        pltpu.make_async_copy(w2_hbm.at[e], w2_buf.at[slot],
                              sem2.at[slot]).wait()

    @pl.when(bu_ref[b] == 1)
    def _():
        e = rex_ref[ri]
        xs = xs_ref[...].astype(jnp.bfloat16)
        hh = _H // 2
        o = jnp.zeros((_BLOCK, _D), jnp.float32) + b2_ref[e][None, :]
        for j in range(2):
            w1h = w1_buf[slot, pl.ds(j * hh, hh), :].astype(jnp.bfloat16)
            hv = lax.dot_general(xs, w1h, (((1,), (1,)), ((), ())),
                                 preferred_element_type=jnp.float32)
            hv = jnp.maximum(hv + b1_ref[e, pl.ds(j * hh, hh)][None, :], 0.0)
            w2h = w2_buf[slot, :, pl.ds(j * hh, hh)].astype(jnp.bfloat16)
            o = o + lax.dot_general(hv.astype(jnp.bfloat16), w2h,
                                    (((1,), (1,)), ((), ())),
                                    preferred_element_type=jnp.float32)
        mx = jnp.max(o, axis=1, keepdims=True)
        ex = jnp.exp(o - mx)
        out_ref[...] = ex / jnp.sum(ex, axis=1, keepdims=True)


def _ffn(xs, W1, b1, W2, b2, ri, first, rex, nr, bu):
    grid_spec = pltpu.PrefetchScalarGridSpec(
        num_scalar_prefetch=5,
        grid=(_NBLK,),
        in_specs=[
            pl.BlockSpec((_BLOCK, _D), lambda b, *_: (b, 0)),
            pl.BlockSpec(memory_space=pl.ANY),
            pl.BlockSpec((_E, _H), lambda b, *_: (0, 0)),
            pl.BlockSpec(memory_space=pl.ANY),
            pl.BlockSpec((_E, _D), lambda b, *_: (0, 0)),
        ],
        out_specs=pl.BlockSpec((_BLOCK, _D), lambda b, *_: (b, 0)),
        scratch_shapes=[
            pltpu.VMEM((2, _H, _D), jnp.float32),
            pltpu.VMEM((2, _D, _H), jnp.float32),
            pltpu.SemaphoreType.DMA((2,)),
            pltpu.SemaphoreType.DMA((2,)),
        ],
    )
    return pl.pallas_call(
        _ffn_body,
        grid_spec=grid_spec,
        out_shape=jax.ShapeDtypeStruct((_PAD, _D), jnp.float32),
        compiler_params=pltpu.CompilerParams(
            vmem_limit_bytes=112 * 1024 * 1024),
    )(ri, first, rex, nr, bu, xs, W1, b1, W2, b2)


def _combine_gather(contrib, p):
    ch = _N // _NW
    mesh = plsc.VectorSubcoreMesh(core_axis_name="c", subcore_axis_name="s")

    @functools.partial(
        pl.kernel, mesh=mesh,
        out_type=(
            jax.ShapeDtypeStruct((_N, _D), jnp.float32),
            jax.ShapeDtypeStruct((_N, _D), jnp.float32),
        ),
        scratch_types=[
            pltpu.VMEM((ch,), jnp.int32),
            pltpu.VMEM((ch,), jnp.int32),
            pltpu.VMEM((ch, _D), jnp.float32),
            pltpu.VMEM((ch, _D), jnp.float32),
            pltpu.SemaphoreType.DMA,
            pltpu.SemaphoreType.DMA,
        ],
    )
    def k(contrib_hbm, p_hbm, c1_hbm, c2_hbm, i1v, i2v, r1v, r2v, s1, s2):
        wid = lax.axis_index("s") * 2 + lax.axis_index("c")
        base = wid * ch
        pltpu.sync_copy(p_hbm.at[pl.ds(base, ch)], i1v)
        pltpu.sync_copy(p_hbm.at[pl.ds(_N + base, ch)], i2v)
        cp1 = pltpu.async_copy(contrib_hbm.at[i1v], r1v, s1)
        cp2 = pltpu.async_copy(contrib_hbm.at[i2v], r2v, s2)
        cp1.wait()
        w1 = pltpu.make_async_copy(r1v, c1_hbm.at[pl.ds(base, ch)], s1)
        w1.start()
        cp2.wait()
        w2 = pltpu.make_async_copy(r2v, c2_hbm.at[pl.ds(base, ch)], s2)
        w2.start()
        w1.wait()
        w2.wait()

    return k(contrib, p)


def _combine_body(c1_ref, c2_ref, g1_ref, g2_ref, out_ref):
    c = g1_ref[...] * c1_ref[...] + g2_ref[...] * c2_ref[...]
    c = jnp.where(c == 0.0, jnp.float32(_EPS), c)
    out_ref[...] = jnp.log(c)


def _combine(c1, c2, g):
    nb = _N // _BLOCK
    return pl.pallas_call(
        _combine_body,
        grid=(nb,),
        in_specs=[
            pl.BlockSpec((_BLOCK, _D), lambda i: (i, 0)),
            pl.BlockSpec((_BLOCK, _D), lambda i: (i, 0)),
            pl.BlockSpec((_BLOCK, 1), lambda i: (i, 0)),
            pl.BlockSpec((_BLOCK, 1), lambda i: (nb + i, 0)),
        ],
        out_specs=pl.BlockSpec((_BLOCK, _D), lambda i: (i, 0)),
        out_shape=jax.ShapeDtypeStruct((_N, _D), jnp.float32),
    )(c1, c2, g, g)


def kernel(x, w_gate, W1, b1, W2, b2):
    p2, g2_, ri2, first2, rex2, nr2, bu2 = _gating(x, w_gate)
    p = p2.reshape((_NPAIR,))
    ri = ri2.reshape((_NBLK,))
    first = first2.reshape((_NBLK,))
    rex = rex2.reshape((_E,))
    nr = nr2.reshape((1,))
    bu = bu2.reshape((_NBLK,))
    xs = _dispatch(x, p)
    contrib = _ffn(xs, W1, b1, W2, b2, ri, first, rex, nr, bu)
    c1, c2 = _combine_gather(contrib, p)
    return _combine(c1, c2, g2_)


# speedup vs baseline: 1.3889x; 1.3889x over previous
import functools

import numpy as np
import jax
import jax.numpy as jnp
from jax import lax
from jax.experimental import pallas as pl
from jax.experimental.pallas import tpu as pltpu
from jax.experimental.pallas import tpu_sc as plsc

_N, _D, _H, _E, _K = 2048, 768, 3072, 8, 2
_BLOCK = 512
_NBLK = (_N * _K) // _BLOCK + _E
_PAD = _NBLK * _BLOCK
_NPAIR = _N * _K
_EPS = float(np.finfo(np.float64).eps)
_NW = 32


def _gate_body(x_ref, wg_ref, p_ref, g_ref, ri_ref, first_ref, rex_ref,
               nr_ref, bu_ref):
    x = x_ref[...]
    wg = wg_ref[...]
    logits = lax.dot_general(x, wg, (((1,), (1,)), ((), ())),
                             preferred_element_type=jnp.float32)
    ioe = lax.broadcasted_iota(jnp.int32, (_N, _E), 1)
    m1 = jnp.max(logits, axis=1, keepdims=True)
    i1 = jnp.min(jnp.where(logits == m1, ioe, _E), axis=1, keepdims=True)
    l2 = jnp.where(ioe == i1, jnp.float32(-jnp.inf), logits)
    m2 = jnp.max(l2, axis=1, keepdims=True)
    i2 = jnp.min(jnp.where(l2 == m2, ioe, _E), axis=1, keepdims=True)
    e21 = jnp.exp(m2 - m1)
    g1 = 1.0 / (1.0 + e21)
    g2 = e21 / (1.0 + e21)

    oh1 = (ioe == i1).astype(jnp.float32)
    oh2 = (ioe == i2).astype(jnp.float32)
    m_oh = jnp.concatenate([oh1, oh2], axis=0)

    bs = 512
    ti = lax.broadcasted_iota(jnp.int32, (bs, bs), 0)
    tj = lax.broadcasted_iota(jnp.int32, (bs, bs), 1)
    tri = (ti > tj).astype(jnp.float32)
    run = jnp.zeros((1, _E), jnp.float32)
    rank_rows = []
    for j in range(_NPAIR // bs):
        mj = m_oh[j * bs:(j + 1) * bs]
        rank_rows.append(
            lax.dot_general(tri, mj, (((1,), (0,)), ((), ())),
                            preferred_element_type=jnp.float32) + run)
        run = run + jnp.sum(mj, axis=0, keepdims=True)
    ranks = jnp.concatenate(rank_rows, axis=0)
    counts = run

    nb = jnp.floor((counts + (_BLOCK - 1)) / _BLOCK)
    si = lax.broadcasted_iota(jnp.int32, (_E, _E), 0)
    sj = lax.broadcasted_iota(jnp.int32, (_E, _E), 1)
    sl = (si < sj).astype(jnp.float32)
    bo = lax.dot_general(nb, sl, (((1,), (0,)), ((), ())),
                         preferred_element_type=jnp.float32)
    cnb = bo + nb

    slot_base = lax.dot_general(m_oh, bo * _BLOCK, (((1,), (1,)), ((), ())),
                                preferred_element_type=jnp.float32)
    rank_r = jnp.sum(ranks * m_oh, axis=1, keepdims=True)
    p_ref[...] = (slot_base + rank_r).astype(jnp.int32)
    g_ref[...] = jnp.concatenate([g1, g2], axis=0)

    iobi = lax.broadcasted_iota(jnp.int32, (_NBLK, _E), 0)
    nbpos = nb > 0
    first_m = jnp.logical_and(iobi == bo.astype(jnp.int32),
                              jnp.broadcast_to(nbpos, (_NBLK, _E)))
    first = jnp.sum(first_m.astype(jnp.float32), axis=1, keepdims=True)
    t24i = lax.broadcasted_iota(jnp.int32, (_NBLK, _NBLK), 0)
    t24j = lax.broadcasted_iota(jnp.int32, (_NBLK, _NBLK), 1)
    t24 = (t24i >= t24j).astype(jnp.float32)
    ri = lax.dot_general(t24, first, (((1,), (0,)), ((), ())),
                         preferred_element_type=jnp.float32) - 1.0
    ridx = lax.dot_general(nbpos.astype(jnp.float32), sl,
                           (((1,), (0,)), ((), ())),
                           preferred_element_type=jnp.float32)
    iorr = lax.broadcasted_iota(jnp.int32, (_E, _E), 0)
    run_m = jnp.logical_and(iorr == ridx.astype(jnp.int32),
                            jnp.broadcast_to(nbpos, (_E, _E)))
    lanecol = lax.broadcasted_iota(jnp.int32, (_E, 1), 0).astype(jnp.float32)
    rex = lax.dot_general(run_m.astype(jnp.float32), lanecol,
                          (((1,), (0,)), ((), ())),
                          preferred_element_type=jnp.float32)
    nr = jnp.sum(nbpos.astype(jnp.float32), axis=1, keepdims=True)
    ri_ref[...] = ri.astype(jnp.int32)
    first_ref[...] = first.astype(jnp.int32)
    rex_ref[...] = rex.astype(jnp.int32)
    nr_ref[...] = nr.astype(jnp.int32)
    total = jnp.sum(nb, axis=1, keepdims=True)
    iob1 = lax.broadcasted_iota(jnp.int32, (_NBLK, 1), 0).astype(jnp.float32)
    bu_ref[...] = (iob1 < total).astype(jnp.int32)


def _gating(x, w_gate):
    return pl.pallas_call(
        _gate_body,
        out_shape=(
            jax.ShapeDtypeStruct((_NPAIR, 1), jnp.int32),
            jax.ShapeDtypeStruct((_NPAIR, 1), jnp.float32),
            jax.ShapeDtypeStruct((_NBLK, 1), jnp.int32),
            jax.ShapeDtypeStruct((_NBLK, 1), jnp.int32),
            jax.ShapeDtypeStruct((_E, 1), jnp.int32),
            jax.ShapeDtypeStruct((1, 1), jnp.int32),
            jax.ShapeDtypeStruct((_NBLK, 1), jnp.int32),
        ),
    )(x, w_gate)


def _dispatch(x, p):
    ch = _NPAIR // _NW
    hc = ch // 2
    mesh = plsc.VectorSubcoreMesh(core_axis_name="c", subcore_axis_name="s")

    @functools.partial(
        pl.kernel, mesh=mesh,
        out_type=jax.ShapeDtypeStruct((_PAD, _D), jnp.float32),
        scratch_types=[
            pltpu.VMEM((2, hc), jnp.int32),
            pltpu.VMEM((2, hc, _D), jnp.float32),
            pltpu.SemaphoreType.DMA((2,)),
            pltpu.SemaphoreType.DMA((2,)),
        ],
    )
    def k(x_hbm, p_hbm, xs_hbm, idx_v, rows_v, lsem, ssem):
        wid = lax.axis_index("s") * 2 + lax.axis_index("c")
        base = wid * ch
        xb = lax.rem(base, _N)
        pltpu.sync_copy(p_hbm.at[pl.ds(base, hc)], idx_v.at[0])
        pltpu.sync_copy(p_hbm.at[pl.ds(base + hc, hc)], idx_v.at[1])
        l0 = pltpu.make_async_copy(x_hbm.at[pl.ds(xb, hc)],
                                   rows_v.at[0], lsem.at[0])
        l1 = pltpu.make_async_copy(x_hbm.at[pl.ds(xb + hc, hc)],
                                   rows_v.at[1], lsem.at[1])
        l0.start()
        l1.start()
        l0.wait()
        s0 = pltpu.make_async_copy(rows_v.at[0], xs_hbm.at[idx_v.at[0]],
                                   ssem.at[0])
        s0.start()
        l1.wait()
        s1 = pltpu.make_async_copy(rows_v.at[1], xs_hbm.at[idx_v.at[1]],
                                   ssem.at[1])
        s1.start()
        s0.wait()
        s1.wait()

    return k(x, p)


def _ffn_body(ri_ref, first_ref, rex_ref, nr_ref, bu_ref,
              xs_ref, w1_hbm, b1_ref, w2_hbm, b2_ref, out_ref,
              w1_buf, w2_buf, sem1, sem2):
    b = pl.program_id(0)
    ri = ri_ref[b]
    slot = lax.rem(ri, 2)

    def _start_fetch(run, slot_):
        e = rex_ref[run]
        pltpu.make_async_copy(w1_hbm.at[e], w1_buf.at[slot_],
                              sem1.at[slot_]).start()
        pltpu.make_async_copy(w2_hbm.at[e], w2_buf.at[slot_],
                              sem2.at[slot_]).start()

    @pl.when(b == 0)
    def _():
        _start_fetch(0, 0)

    @pl.when(first_ref[b] == 1)
    def _():
        nxt = ri + 1

        @pl.when(nxt < nr_ref[0])
        def _():
            _start_fetch(nxt, lax.rem(nxt, 2))

        e = rex_ref[ri]
        pltpu.make_async_copy(w1_hbm.at[e], w1_buf.at[slot],
                              sem1.at[slot]).wait()
        pltpu.make_async_copy(w2_hbm.at[e], w2_buf.at[slot],
                              sem2.at[slot]).wait()

    @pl.when(bu_ref[b] == 1)
    def _():
        e = rex_ref[ri]
        xs = xs_ref[...].astype(jnp.bfloat16)
        hh = _H // 2
        o = jnp.zeros((_BLOCK, _D), jnp.float32) + b2_ref[e][None, :]
        for j in range(2):
            w1h = w1_buf[slot, pl.ds(j * hh, hh), :].astype(jnp.bfloat16)
            hv = lax.dot_general(xs, w1h, (((1,), (1,)), ((), ())),
                                 preferred_element_type=jnp.float32)
            hv = jnp.maximum(hv + b1_ref[e, pl.ds(j * hh, hh)][None, :], 0.0)
            w2h = w2_buf[slot, :, pl.ds(j * hh, hh)].astype(jnp.bfloat16)
            o = o + lax.dot_general(hv.astype(jnp.bfloat16), w2h,
                                    (((1,), (1,)), ((), ())),
                                    preferred_element_type=jnp.float32)
        mx = jnp.max(o, axis=1, keepdims=True)
        ex = jnp.exp(o - mx)
        out_ref[...] = ex / jnp.sum(ex, axis=1, keepdims=True)


def _ffn(xs, W1, b1, W2, b2, ri, first, rex, nr, bu):
    grid_spec = pltpu.PrefetchScalarGridSpec(
        num_scalar_prefetch=5,
        grid=(_NBLK,),
        in_specs=[
            pl.BlockSpec((_BLOCK, _D), lambda b, *_: (b, 0)),
            pl.BlockSpec(memory_space=pl.ANY),
            pl.BlockSpec((_E, _H), lambda b, *_: (0, 0)),
            pl.BlockSpec(memory_space=pl.ANY),
            pl.BlockSpec((_E, _D), lambda b, *_: (0, 0)),
        ],
        out_specs=pl.BlockSpec((_BLOCK, _D), lambda b, *_: (b, 0)),
        scratch_shapes=[
            pltpu.VMEM((2, _H, _D), jnp.float32),
            pltpu.VMEM((2, _D, _H), jnp.float32),
            pltpu.SemaphoreType.DMA((2,)),
            pltpu.SemaphoreType.DMA((2,)),
        ],
    )
    return pl.pallas_call(
        _ffn_body,
        grid_spec=grid_spec,
        out_shape=jax.ShapeDtypeStruct((_PAD, _D), jnp.float32),
        compiler_params=pltpu.CompilerParams(
            vmem_limit_bytes=112 * 1024 * 1024),
    )(ri, first, rex, nr, bu, xs, W1, b1, W2, b2)


def _combine_gather(contrib, p):
    ch = _N // _NW
    mesh = plsc.VectorSubcoreMesh(core_axis_name="c", subcore_axis_name="s")

    @functools.partial(
        pl.kernel, mesh=mesh,
        out_type=(
            jax.ShapeDtypeStruct((_N, _D), jnp.float32),
            jax.ShapeDtypeStruct((_N, _D), jnp.float32),
        ),
        scratch_types=[
            pltpu.VMEM((ch,), jnp.int32),
            pltpu.VMEM((ch,), jnp.int32),
            pltpu.VMEM((ch, _D), jnp.float32),
            pltpu.VMEM((ch, _D), jnp.float32),
            pltpu.SemaphoreType.DMA,
            pltpu.SemaphoreType.DMA,
        ],
    )
    def k(contrib_hbm, p_hbm, c1_hbm, c2_hbm, i1v, i2v, r1v, r2v, s1, s2):
        wid = lax.axis_index("s") * 2 + lax.axis_index("c")
        base = wid * ch
        pltpu.sync_copy(p_hbm.at[pl.ds(base, ch)], i1v)
        pltpu.sync_copy(p_hbm.at[pl.ds(_N + base, ch)], i2v)
        cp1 = pltpu.async_copy(contrib_hbm.at[i1v], r1v, s1)
        cp2 = pltpu.async_copy(contrib_hbm.at[i2v], r2v, s2)
        cp1.wait()
        w1 = pltpu.make_async_copy(r1v, c1_hbm.at[pl.ds(base, ch)], s1)
        w1.start()
        cp2.wait()
        w2 = pltpu.make_async_copy(r2v, c2_hbm.at[pl.ds(base, ch)], s2)
        w2.start()
        w1.wait()
        w2.wait()

    return k(contrib, p)


def _combine_body(c1_ref, c2_ref, g1_ref, g2_ref, out_ref):
    c = g1_ref[...] * c1_ref[...] + g2_ref[...] * c2_ref[...]
    c = jnp.where(c == 0.0, jnp.float32(_EPS), c)
    out_ref[...] = jnp.log(c)


def _combine(c1, c2, g):
    nb = _N // _BLOCK
    return pl.pallas_call(
        _combine_body,
        grid=(nb,),
        in_specs=[
            pl.BlockSpec((_BLOCK, _D), lambda i: (i, 0)),
            pl.BlockSpec((_BLOCK, _D), lambda i: (i, 0)),
            pl.BlockSpec((_BLOCK, 1), lambda i: (i, 0)),
            pl.BlockSpec((_BLOCK, 1), lambda i: (nb + i, 0)),
        ],
        out_specs=pl.BlockSpec((_BLOCK, _D), lambda i: (i, 0)),
        out_shape=jax.ShapeDtypeStruct((_N, _D), jnp.float32),
    )(c1, c2, g, g)


def kernel(x, w_gate, W1, b1, W2, b2):
    p2, g2_, ri2, first2, rex2, nr2, bu2 = _gating(x, w_gate)
    p = p2.reshape((_NPAIR,))
    ri = ri2.reshape((_NBLK,))
    first = first2.reshape((_NBLK,))
    rex = rex2.reshape((_E,))
    nr = nr2.reshape((1,))
    bu = bu2.reshape((_NBLK,))
    xs = _dispatch(x, p)
    contrib = _ffn(xs, W1, b1, W2, b2, ri, first, rex, nr, bu)
    c1, c2 = _combine_gather(contrib, p)
    return _combine(c1, c2, g2_)


# speedup vs baseline: 1.3999x; 1.0079x over previous
import functools

import numpy as np
import jax
import jax.numpy as jnp
from jax import lax
from jax.experimental import pallas as pl
from jax.experimental.pallas import tpu as pltpu
from jax.experimental.pallas import tpu_sc as plsc

_N, _D, _H, _E, _K = 2048, 768, 3072, 8, 2
_BLOCK = 256
_NBLK = (_N * _K) // _BLOCK + _E
_PAD = _NBLK * _BLOCK
_NPAIR = _N * _K
_EPS = float(np.finfo(np.float64).eps)
_NW = 32


def _gate_body(x_ref, wg_ref, p_ref, g_ref, ri_ref, first_ref, rex_ref,
               nr_ref, bu_ref):
    x = x_ref[...]
    wg = wg_ref[...]
    logits = lax.dot_general(x, wg, (((1,), (1,)), ((), ())),
                             preferred_element_type=jnp.float32)
    ioe = lax.broadcasted_iota(jnp.int32, (_N, _E), 1)
    m1 = jnp.max(logits, axis=1, keepdims=True)
    i1 = jnp.min(jnp.where(logits == m1, ioe, _E), axis=1, keepdims=True)
    l2 = jnp.where(ioe == i1, jnp.float32(-jnp.inf), logits)
    m2 = jnp.max(l2, axis=1, keepdims=True)
    i2 = jnp.min(jnp.where(l2 == m2, ioe, _E), axis=1, keepdims=True)
    e21 = jnp.exp(m2 - m1)
    g1 = 1.0 / (1.0 + e21)
    g2 = e21 / (1.0 + e21)

    oh1 = (ioe == i1).astype(jnp.float32)
    oh2 = (ioe == i2).astype(jnp.float32)
    m_oh = jnp.concatenate([oh1, oh2], axis=0)

    bs = 512
    ti = lax.broadcasted_iota(jnp.int32, (bs, bs), 0)
    tj = lax.broadcasted_iota(jnp.int32, (bs, bs), 1)
    tri = (ti > tj).astype(jnp.float32)
    run = jnp.zeros((1, _E), jnp.float32)
    rank_rows = []
    for j in range(_NPAIR // bs):
        mj = m_oh[j * bs:(j + 1) * bs]
        rank_rows.append(
            lax.dot_general(tri, mj, (((1,), (0,)), ((), ())),
                            preferred_element_type=jnp.float32) + run)
        run = run + jnp.sum(mj, axis=0, keepdims=True)
    ranks = jnp.concatenate(rank_rows, axis=0)
    counts = run

    nb = jnp.floor((counts + (_BLOCK - 1)) / _BLOCK)
    si = lax.broadcasted_iota(jnp.int32, (_E, _E), 0)
    sj = lax.broadcasted_iota(jnp.int32, (_E, _E), 1)
    sl = (si < sj).astype(jnp.float32)
    bo = lax.dot_general(nb, sl, (((1,), (0,)), ((), ())),
                         preferred_element_type=jnp.float32)
    cnb = bo + nb

    slot_base = lax.dot_general(m_oh, bo * _BLOCK, (((1,), (1,)), ((), ())),
                                preferred_element_type=jnp.float32)
    rank_r = jnp.sum(ranks * m_oh, axis=1, keepdims=True)
    p_ref[...] = (slot_base + rank_r).astype(jnp.int32)
    g_ref[...] = jnp.concatenate([g1, g2], axis=0)

    iobi = lax.broadcasted_iota(jnp.int32, (_NBLK, _E), 0)
    nbpos = nb > 0
    first_m = jnp.logical_and(iobi == bo.astype(jnp.int32),
                              jnp.broadcast_to(nbpos, (_NBLK, _E)))
    first = jnp.sum(first_m.astype(jnp.float32), axis=1, keepdims=True)
    t24i = lax.broadcasted_iota(jnp.int32, (_NBLK, _NBLK), 0)
    t24j = lax.broadcasted_iota(jnp.int32, (_NBLK, _NBLK), 1)
    t24 = (t24i >= t24j).astype(jnp.float32)
    ri = lax.dot_general(t24, first, (((1,), (0,)), ((), ())),
                         preferred_element_type=jnp.float32) - 1.0
    ridx = lax.dot_general(nbpos.astype(jnp.float32), sl,
                           (((1,), (0,)), ((), ())),
                           preferred_element_type=jnp.float32)
    iorr = lax.broadcasted_iota(jnp.int32, (_E, _E), 0)
    run_m = jnp.logical_and(iorr == ridx.astype(jnp.int32),
                            jnp.broadcast_to(nbpos, (_E, _E)))
    lanecol = lax.broadcasted_iota(jnp.int32, (_E, 1), 0).astype(jnp.float32)
    rex = lax.dot_general(run_m.astype(jnp.float32), lanecol,
                          (((1,), (0,)), ((), ())),
                          preferred_element_type=jnp.float32)
    nr = jnp.sum(nbpos.astype(jnp.float32), axis=1, keepdims=True)
    ri_ref[...] = ri.astype(jnp.int32)
    first_ref[...] = first.astype(jnp.int32)
    rex_ref[...] = rex.astype(jnp.int32)
    nr_ref[...] = nr.astype(jnp.int32)
    total = jnp.sum(nb, axis=1, keepdims=True)
    iob1 = lax.broadcasted_iota(jnp.int32, (_NBLK, 1), 0).astype(jnp.float32)
    bu_ref[...] = (iob1 < total).astype(jnp.int32)


def _gating(x, w_gate):
    return pl.pallas_call(
        _gate_body,
        out_shape=(
            jax.ShapeDtypeStruct((_NPAIR, 1), jnp.int32),
            jax.ShapeDtypeStruct((_NPAIR, 1), jnp.float32),
            jax.ShapeDtypeStruct((_NBLK, 1), jnp.int32),
            jax.ShapeDtypeStruct((_NBLK, 1), jnp.int32),
            jax.ShapeDtypeStruct((_E, 1), jnp.int32),
            jax.ShapeDtypeStruct((1, 1), jnp.int32),
            jax.ShapeDtypeStruct((_NBLK, 1), jnp.int32),
        ),
    )(x, w_gate)


def _dispatch(x, p):
    ch = _NPAIR // _NW
    hc = ch // 2
    mesh = plsc.VectorSubcoreMesh(core_axis_name="c", subcore_axis_name="s")

    @functools.partial(
        pl.kernel, mesh=mesh,
        out_type=jax.ShapeDtypeStruct((_PAD, _D), jnp.float32),
        scratch_types=[
            pltpu.VMEM((2, hc), jnp.int32),
            pltpu.VMEM((2, hc, _D), jnp.float32),
            pltpu.SemaphoreType.DMA((2,)),
            pltpu.SemaphoreType.DMA((2,)),
        ],
    )
    def k(x_hbm, p_hbm, xs_hbm, idx_v, rows_v, lsem, ssem):
        wid = lax.axis_index("s") * 2 + lax.axis_index("c")
        base = wid * ch
        xb = lax.rem(base, _N)
        pltpu.sync_copy(p_hbm.at[pl.ds(base, hc)], idx_v.at[0])
        pltpu.sync_copy(p_hbm.at[pl.ds(base + hc, hc)], idx_v.at[1])
        l0 = pltpu.make_async_copy(x_hbm.at[pl.ds(xb, hc)],
                                   rows_v.at[0], lsem.at[0])
        l1 = pltpu.make_async_copy(x_hbm.at[pl.ds(xb + hc, hc)],
                                   rows_v.at[1], lsem.at[1])
        l0.start()
        l1.start()
        l0.wait()
        s0 = pltpu.make_async_copy(rows_v.at[0], xs_hbm.at[idx_v.at[0]],
                                   ssem.at[0])
        s0.start()
        l1.wait()
        s1 = pltpu.make_async_copy(rows_v.at[1], xs_hbm.at[idx_v.at[1]],
                                   ssem.at[1])
        s1.start()
        s0.wait()
        s1.wait()

    return k(x, p)


def _ffn_body(ri_ref, first_ref, rex_ref, nr_ref, bu_ref,
              xs_ref, w1_hbm, b1_ref, w2_hbm, b2_ref, out_ref,
              w1_buf, w2_buf, w1_bf, w2_bf, sem1, sem2):
    b = pl.program_id(0)
    ri = ri_ref[b]
    slot = lax.rem(ri, 2)

    def _start_fetch(run, slot_):
        e = rex_ref[run]
        pltpu.make_async_copy(w1_hbm.at[e], w1_buf.at[slot_],
                              sem1.at[slot_]).start()
        pltpu.make_async_copy(w2_hbm.at[e], w2_buf.at[slot_],
                              sem2.at[slot_]).start()

    @pl.when(b == 0)
    def _():
        _start_fetch(0, 0)

    @pl.when(first_ref[b] == 1)
    def _():
        nxt = ri + 1

        @pl.when(nxt < nr_ref[0])
        def _():
            _start_fetch(nxt, lax.rem(nxt, 2))

        e = rex_ref[ri]
        pltpu.make_async_copy(w1_hbm.at[e], w1_buf.at[slot],
                              sem1.at[slot]).wait()
        pltpu.make_async_copy(w2_hbm.at[e], w2_buf.at[slot],
                              sem2.at[slot]).wait()
        hh_ = _H // 2
        for j in range(2):
            w1_bf[slot, pl.ds(j * hh_, hh_), :] = (
                w1_buf[slot, pl.ds(j * hh_, hh_), :].astype(jnp.bfloat16))
            w2_bf[slot, :, pl.ds(j * hh_, hh_)] = (
                w2_buf[slot, :, pl.ds(j * hh_, hh_)].astype(jnp.bfloat16))

    @pl.when(bu_ref[b] == 1)
    def _():
        e = rex_ref[ri]
        xs = xs_ref[...].astype(jnp.bfloat16)
        hh = _H // 2
        o = jnp.zeros((_BLOCK, _D), jnp.float32) + b2_ref[e][None, :]
        for j in range(2):
            w1h = w1_bf[slot, pl.ds(j * hh, hh), :]
            hv = lax.dot_general(xs, w1h, (((1,), (1,)), ((), ())),
                                 preferred_element_type=jnp.float32)
            hv = jnp.maximum(hv + b1_ref[e, pl.ds(j * hh, hh)][None, :], 0.0)
            w2h = w2_bf[slot, :, pl.ds(j * hh, hh)]
            o = o + lax.dot_general(hv.astype(jnp.bfloat16), w2h,
                                    (((1,), (1,)), ((), ())),
                                    preferred_element_type=jnp.float32)
        mx = jnp.max(o, axis=1, keepdims=True)
        ex = jnp.exp(o - mx)
        out_ref[...] = ex / jnp.sum(ex, axis=1, keepdims=True)


def _ffn(xs, W1, b1, W2, b2, ri, first, rex, nr, bu):
    grid_spec = pltpu.PrefetchScalarGridSpec(
        num_scalar_prefetch=5,
        grid=(_NBLK,),
        in_specs=[
            pl.BlockSpec((_BLOCK, _D), lambda b, *_: (b, 0)),
            pl.BlockSpec(memory_space=pl.ANY),
            pl.BlockSpec((_E, _H), lambda b, *_: (0, 0)),
            pl.BlockSpec(memory_space=pl.ANY),
            pl.BlockSpec((_E, _D), lambda b, *_: (0, 0)),
        ],
        out_specs=pl.BlockSpec((_BLOCK, _D), lambda b, *_: (b, 0)),
        scratch_shapes=[
            pltpu.VMEM((2, _H, _D), jnp.float32),
            pltpu.VMEM((2, _D, _H), jnp.float32),
            pltpu.VMEM((2, _H, _D), jnp.bfloat16),
            pltpu.VMEM((2, _D, _H), jnp.bfloat16),
            pltpu.SemaphoreType.DMA((2,)),
            pltpu.SemaphoreType.DMA((2,)),
        ],
    )
    return pl.pallas_call(
        _ffn_body,
        grid_spec=grid_spec,
        out_shape=jax.ShapeDtypeStruct((_PAD, _D), jnp.float32),
        compiler_params=pltpu.CompilerParams(
            vmem_limit_bytes=112 * 1024 * 1024),
    )(ri, first, rex, nr, bu, xs, W1, b1, W2, b2)


def _combine_gather(contrib, p):
    ch = _N // _NW
    mesh = plsc.VectorSubcoreMesh(core_axis_name="c", subcore_axis_name="s")

    @functools.partial(
        pl.kernel, mesh=mesh,
        out_type=(
            jax.ShapeDtypeStruct((_N, _D), jnp.float32),
            jax.ShapeDtypeStruct((_N, _D), jnp.float32),
        ),
        scratch_types=[
            pltpu.VMEM((ch,), jnp.int32),
            pltpu.VMEM((ch,), jnp.int32),
            pltpu.VMEM((ch, _D), jnp.float32),
            pltpu.VMEM((ch, _D), jnp.float32),
            pltpu.SemaphoreType.DMA,
            pltpu.SemaphoreType.DMA,
        ],
    )
    def k(contrib_hbm, p_hbm, c1_hbm, c2_hbm, i1v, i2v, r1v, r2v, s1, s2):
        wid = lax.axis_index("s") * 2 + lax.axis_index("c")
        base = wid * ch
        pltpu.sync_copy(p_hbm.at[pl.ds(base, ch)], i1v)
        pltpu.sync_copy(p_hbm.at[pl.ds(_N + base, ch)], i2v)
        cp1 = pltpu.async_copy(contrib_hbm.at[i1v], r1v, s1)
        cp2 = pltpu.async_copy(contrib_hbm.at[i2v], r2v, s2)
        cp1.wait()
        w1 = pltpu.make_async_copy(r1v, c1_hbm.at[pl.ds(base, ch)], s1)
        w1.start()
        cp2.wait()
        w2 = pltpu.make_async_copy(r2v, c2_hbm.at[pl.ds(base, ch)], s2)
        w2.start()
        w1.wait()
        w2.wait()

    return k(contrib, p)


def _combine_body(c1_ref, c2_ref, g1_ref, g2_ref, out_ref):
    c = g1_ref[...] * c1_ref[...] + g2_ref[...] * c2_ref[...]
    c = jnp.where(c == 0.0, jnp.float32(_EPS), c)
    out_ref[...] = jnp.log(c)


def _combine(c1, c2, g):
    nb = _N // _BLOCK
    return pl.pallas_call(
        _combine_body,
        grid=(nb,),
        in_specs=[
            pl.BlockSpec((_BLOCK, _D), lambda i: (i, 0)),
            pl.BlockSpec((_BLOCK, _D), lambda i: (i, 0)),
            pl.BlockSpec((_BLOCK, 1), lambda i: (i, 0)),
            pl.BlockSpec((_BLOCK, 1), lambda i: (nb + i, 0)),
        ],
        out_specs=pl.BlockSpec((_BLOCK, _D), lambda i: (i, 0)),
        out_shape=jax.ShapeDtypeStruct((_N, _D), jnp.float32),
    )(c1, c2, g, g)


def kernel(x, w_gate, W1, b1, W2, b2):
    p2, g2_, ri2, first2, rex2, nr2, bu2 = _gating(x, w_gate)
    p = p2.reshape((_NPAIR,))
    ri = ri2.reshape((_NBLK,))
    first = first2.reshape((_NBLK,))
    rex = rex2.reshape((_E,))
    nr = nr2.reshape((1,))
    bu = bu2.reshape((_NBLK,))
    xs = _dispatch(x, p)
    contrib = _ffn(xs, W1, b1, W2, b2, ri, first, rex, nr, bu)
    c1, c2 = _combine_gather(contrib, p)
    return _combine(c1, c2, g2_)


# speedup vs baseline: 1.4233x; 1.0167x over previous
import functools

import numpy as np
import jax
import jax.numpy as jnp
from jax import lax
from jax.experimental import pallas as pl
from jax.experimental.pallas import tpu as pltpu
from jax.experimental.pallas import tpu_sc as plsc

_N, _D, _H, _E, _K = 2048, 768, 3072, 8, 2
_BLOCK = 256
_NBLK = (_N * _K) // _BLOCK + _E
_PAD = _NBLK * _BLOCK
_NPAIR = _N * _K
_EPS = float(np.finfo(np.float64).eps)
_NW = 32


def _gate_body(x_ref, wg_ref, p_ref, g_ref, ri_ref, first_ref, rex_ref,
               nr_ref, bu_ref):
    x = x_ref[...]
    wg = wg_ref[...]
    logits = lax.dot_general(x, wg, (((1,), (1,)), ((), ())),
                             preferred_element_type=jnp.float32)
    ioe = lax.broadcasted_iota(jnp.int32, (_N, _E), 1)
    m1 = jnp.max(logits, axis=1, keepdims=True)
    i1 = jnp.min(jnp.where(logits == m1, ioe, _E), axis=1, keepdims=True)
    l2 = jnp.where(ioe == i1, jnp.float32(-jnp.inf), logits)
    m2 = jnp.max(l2, axis=1, keepdims=True)
    i2 = jnp.min(jnp.where(l2 == m2, ioe, _E), axis=1, keepdims=True)
    e21 = jnp.exp(m2 - m1)
    g1 = 1.0 / (1.0 + e21)
    g2 = e21 / (1.0 + e21)

    oh1 = (ioe == i1).astype(jnp.float32)
    oh2 = (ioe == i2).astype(jnp.float32)

    bs = 512
    nchunk = _NPAIR // bs
    ti = lax.broadcasted_iota(jnp.int32, (bs, bs), 0)
    tj = lax.broadcasted_iota(jnp.int32, (bs, bs), 1)
    tri = (ti > tj).astype(jnp.float32)
    run = jnp.zeros((1, _E), jnp.float32)
    rank_chunks = []
    for j in range(nchunk):
        if j < nchunk // 2:
            mj = oh1[j * bs:(j + 1) * bs]
        else:
            mj = oh2[(j - nchunk // 2) * bs:(j - nchunk // 2 + 1) * bs]
        rank_chunks.append(
            lax.dot_general(tri, mj, (((1,), (0,)), ((), ())),
                            preferred_element_type=jnp.float32) + run)
        run = run + jnp.sum(mj, axis=0, keepdims=True)
    counts = run

    nb = jnp.floor((counts + (_BLOCK - 1)) / _BLOCK)
    si = lax.broadcasted_iota(jnp.int32, (_E, _E), 0)
    sj = lax.broadcasted_iota(jnp.int32, (_E, _E), 1)
    sl = (si < sj).astype(jnp.float32)
    bo = lax.dot_general(nb, sl, (((1,), (0,)), ((), ())),
                         preferred_element_type=jnp.float32)
    cnb = bo + nb

    p_chunks = []
    for j in range(nchunk):
        if j < nchunk // 2:
            mj = oh1[j * bs:(j + 1) * bs]
        else:
            mj = oh2[(j - nchunk // 2) * bs:(j - nchunk // 2 + 1) * bs]
        sb = lax.dot_general(mj, bo * _BLOCK, (((1,), (1,)), ((), ())),
                             preferred_element_type=jnp.float32)
        rr = jnp.sum(rank_chunks[j] * mj, axis=1, keepdims=True)
        p_chunks.append(sb + rr)
    p_ref[...] = jnp.concatenate(p_chunks, axis=0).astype(jnp.int32)
    g_ref[...] = jnp.concatenate([g1, g2], axis=0)

    iobi = lax.broadcasted_iota(jnp.int32, (_NBLK, _E), 0)
    nbpos = nb > 0
    first_m = jnp.logical_and(iobi == bo.astype(jnp.int32),
                              jnp.broadcast_to(nbpos, (_NBLK, _E)))
    first = jnp.sum(first_m.astype(jnp.float32), axis=1, keepdims=True)
    t24i = lax.broadcasted_iota(jnp.int32, (_NBLK, _NBLK), 0)
    t24j = lax.broadcasted_iota(jnp.int32, (_NBLK, _NBLK), 1)
    t24 = (t24i >= t24j).astype(jnp.float32)
    ri = lax.dot_general(t24, first, (((1,), (0,)), ((), ())),
                         preferred_element_type=jnp.float32) - 1.0
    ridx = lax.dot_general(nbpos.astype(jnp.float32), sl,
                           (((1,), (0,)), ((), ())),
                           preferred_element_type=jnp.float32)
    iorr = lax.broadcasted_iota(jnp.int32, (_E, _E), 0)
    run_m = jnp.logical_and(iorr == ridx.astype(jnp.int32),
                            jnp.broadcast_to(nbpos, (_E, _E)))
    lanecol = lax.broadcasted_iota(jnp.int32, (_E, 1), 0).astype(jnp.float32)
    rex = lax.dot_general(run_m.astype(jnp.float32), lanecol,
                          (((1,), (0,)), ((), ())),
                          preferred_element_type=jnp.float32)
    nr = jnp.sum(nbpos.astype(jnp.float32), axis=1, keepdims=True)
    ri_ref[...] = ri.astype(jnp.int32)
    first_ref[...] = first.astype(jnp.int32)
    rex_ref[...] = rex.astype(jnp.int32)
    nr_ref[...] = nr.astype(jnp.int32)
    total = jnp.sum(nb, axis=1, keepdims=True)
    iob1 = lax.broadcasted_iota(jnp.int32, (_NBLK, 1), 0).astype(jnp.float32)
    bu_ref[...] = (iob1 < total).astype(jnp.int32)


def _gating(x, w_gate):
    return pl.pallas_call(
        _gate_body,
        out_shape=(
            jax.ShapeDtypeStruct((_NPAIR, 1), jnp.int32),
            jax.ShapeDtypeStruct((_NPAIR, 1), jnp.float32),
            jax.ShapeDtypeStruct((_NBLK, 1), jnp.int32),
            jax.ShapeDtypeStruct((_NBLK, 1), jnp.int32),
            jax.ShapeDtypeStruct((_E, 1), jnp.int32),
            jax.ShapeDtypeStruct((1, 1), jnp.int32),
            jax.ShapeDtypeStruct((_NBLK, 1), jnp.int32),
        ),
    )(x, w_gate)


def _dispatch(x, p):
    ch = _NPAIR // _NW
    hc = ch // 2
    mesh = plsc.VectorSubcoreMesh(core_axis_name="c", subcore_axis_name="s")

    @functools.partial(
        pl.kernel, mesh=mesh,
        out_type=jax.ShapeDtypeStruct((_PAD, _D), jnp.float32),
        scratch_types=[
            pltpu.VMEM((2, hc), jnp.int32),
            pltpu.VMEM((2, hc, _D), jnp.float32),
            pltpu.SemaphoreType.DMA((2,)),
            pltpu.SemaphoreType.DMA((2,)),
        ],
    )
    def k(x_hbm, p_hbm, xs_hbm, idx_v, rows_v, lsem, ssem):
        wid = lax.axis_index("s") * 2 + lax.axis_index("c")
        base = wid * ch
        xb = lax.rem(base, _N)
        pltpu.sync_copy(p_hbm.at[pl.ds(base, hc)], idx_v.at[0])
        pltpu.sync_copy(p_hbm.at[pl.ds(base + hc, hc)], idx_v.at[1])
        l0 = pltpu.make_async_copy(x_hbm.at[pl.ds(xb, hc)],
                                   rows_v.at[0], lsem.at[0])
        l1 = pltpu.make_async_copy(x_hbm.at[pl.ds(xb + hc, hc)],
                                   rows_v.at[1], lsem.at[1])
        l0.start()
        l1.start()
        l0.wait()
        s0 = pltpu.make_async_copy(rows_v.at[0], xs_hbm.at[idx_v.at[0]],
                                   ssem.at[0])
        s0.start()
        l1.wait()
        s1 = pltpu.make_async_copy(rows_v.at[1], xs_hbm.at[idx_v.at[1]],
                                   ssem.at[1])
        s1.start()
        s0.wait()
        s1.wait()

    return k(x, p)


def _ffn_body(ri_ref, first_ref, rex_ref, nr_ref, bu_ref,
              xs_ref, w1_hbm, b1_ref, w2_hbm, b2_ref, out_ref,
              w1_buf, w2_buf, sem1, sem2):
    b = pl.program_id(0)
    ri = ri_ref[b]
    slot = lax.rem(ri, 2)

    def _start_fetch(run, slot_):
        e = rex_ref[run]
        pltpu.make_async_copy(w1_hbm.at[e], w1_buf.at[slot_],
                              sem1.at[slot_]).start()
        pltpu.make_async_copy(w2_hbm.at[e], w2_buf.at[slot_],
                              sem2.at[slot_]).start()

    @pl.when(b == 0)
    def _():
        _start_fetch(0, 0)

    @pl.when(first_ref[b] == 1)
    def _():
        nxt = ri + 1

        @pl.when(nxt < nr_ref[0])
        def _():
            _start_fetch(nxt, lax.rem(nxt, 2))

        e = rex_ref[ri]
        pltpu.make_async_copy(w1_hbm.at[e], w1_buf.at[slot],
                              sem1.at[slot]).wait()
        pltpu.make_async_copy(w2_hbm.at[e], w2_buf.at[slot],
                              sem2.at[slot]).wait()

    @pl.when(bu_ref[b] == 1)
    def _():
        e = rex_ref[ri]
        xs = xs_ref[...].astype(jnp.bfloat16)
        hh = _H // 2
        o = jnp.zeros((_BLOCK, _D), jnp.float32) + b2_ref[e][None, :]
        for j in range(2):
            w1h = w1_buf[slot, pl.ds(j * hh, hh), :].astype(jnp.bfloat16)
            hv = lax.dot_general(xs, w1h, (((1,), (1,)), ((), ())),
                                 preferred_element_type=jnp.float32)
            hv = jnp.maximum(hv + b1_ref[e, pl.ds(j * hh, hh)][None, :], 0.0)
            w2h = w2_buf[slot, :, pl.ds(j * hh, hh)].astype(jnp.bfloat16)
            o = o + lax.dot_general(hv.astype(jnp.bfloat16), w2h,
                                    (((1,), (1,)), ((), ())),
                                    preferred_element_type=jnp.float32)
        mx = jnp.max(o, axis=1, keepdims=True)
        ex = jnp.exp(o - mx)
        out_ref[...] = ex / jnp.sum(ex, axis=1, keepdims=True)


def _ffn(xs, W1, b1, W2, b2, ri, first, rex, nr, bu):
    grid_spec = pltpu.PrefetchScalarGridSpec(
        num_scalar_prefetch=5,
        grid=(_NBLK,),
        in_specs=[
            pl.BlockSpec((_BLOCK, _D), lambda b, *_: (b, 0)),
            pl.BlockSpec(memory_space=pl.ANY),
            pl.BlockSpec((_E, _H), lambda b, *_: (0, 0)),
            pl.BlockSpec(memory_space=pl.ANY),
            pl.BlockSpec((_E, _D), lambda b, *_: (0, 0)),
        ],
        out_specs=pl.BlockSpec((_BLOCK, _D), lambda b, *_: (b, 0)),
        scratch_shapes=[
            pltpu.VMEM((2, _H, _D), jnp.float32),
            pltpu.VMEM((2, _D, _H), jnp.float32),
            pltpu.SemaphoreType.DMA((2,)),
            pltpu.SemaphoreType.DMA((2,)),
        ],
    )
    return pl.pallas_call(
        _ffn_body,
        grid_spec=grid_spec,
        out_shape=jax.ShapeDtypeStruct((_PAD, _D), jnp.float32),
        compiler_params=pltpu.CompilerParams(
            vmem_limit_bytes=112 * 1024 * 1024),
    )(ri, first, rex, nr, bu, xs, W1, b1, W2, b2)


def _combine_gather(contrib, p):
    ch = _N // _NW
    mesh = plsc.VectorSubcoreMesh(core_axis_name="c", subcore_axis_name="s")

    @functools.partial(
        pl.kernel, mesh=mesh,
        out_type=(
            jax.ShapeDtypeStruct((_N, _D), jnp.float32),
            jax.ShapeDtypeStruct((_N, _D), jnp.float32),
        ),
        scratch_types=[
            pltpu.VMEM((ch,), jnp.int32),
            pltpu.VMEM((ch,), jnp.int32),
            pltpu.VMEM((ch, _D), jnp.float32),
            pltpu.VMEM((ch, _D), jnp.float32),
            pltpu.SemaphoreType.DMA,
            pltpu.SemaphoreType.DMA,
        ],
    )
    def k(contrib_hbm, p_hbm, c1_hbm, c2_hbm, i1v, i2v, r1v, r2v, s1, s2):
        wid = lax.axis_index("s") * 2 + lax.axis_index("c")
        base = wid * ch
        pltpu.sync_copy(p_hbm.at[pl.ds(base, ch)], i1v)
        pltpu.sync_copy(p_hbm.at[pl.ds(_N + base, ch)], i2v)
        cp1 = pltpu.async_copy(contrib_hbm.at[i1v], r1v, s1)
        cp2 = pltpu.async_copy(contrib_hbm.at[i2v], r2v, s2)
        cp1.wait()
        w1 = pltpu.make_async_copy(r1v, c1_hbm.at[pl.ds(base, ch)], s1)
        w1.start()
        cp2.wait()
        w2 = pltpu.make_async_copy(r2v, c2_hbm.at[pl.ds(base, ch)], s2)
        w2.start()
        w1.wait()
        w2.wait()

    return k(contrib, p)


def _combine_body(c1_ref, c2_ref, g1_ref, g2_ref, out_ref):
    c = g1_ref[...] * c1_ref[...] + g2_ref[...] * c2_ref[...]
    c = jnp.where(c == 0.0, jnp.float32(_EPS), c)
    out_ref[...] = jnp.log(c)


def _combine(c1, c2, g):
    nb = _N // _BLOCK
    return pl.pallas_call(
        _combine_body,
        grid=(nb,),
        in_specs=[
            pl.BlockSpec((_BLOCK, _D), lambda i: (i, 0)),
            pl.BlockSpec((_BLOCK, _D), lambda i: (i, 0)),
            pl.BlockSpec((_BLOCK, 1), lambda i: (i, 0)),
            pl.BlockSpec((_BLOCK, 1), lambda i: (nb + i, 0)),
        ],
        out_specs=pl.BlockSpec((_BLOCK, _D), lambda i: (i, 0)),
        out_shape=jax.ShapeDtypeStruct((_N, _D), jnp.float32),
    )(c1, c2, g, g)


def kernel(x, w_gate, W1, b1, W2, b2):
    p2, g2_, ri2, first2, rex2, nr2, bu2 = _gating(x, w_gate)
    p = p2.reshape((_NPAIR,))
    ri = ri2.reshape((_NBLK,))
    first = first2.reshape((_NBLK,))
    rex = rex2.reshape((_E,))
    nr = nr2.reshape((1,))
    bu = bu2.reshape((_NBLK,))
    xs = _dispatch(x, p)
    contrib = _ffn(xs, W1, b1, W2, b2, ri, first, rex, nr, bu)
    c1, c2 = _combine_gather(contrib, p)
    return _combine(c1, c2, g2_)


# speedup vs baseline: 1.4584x; 1.0246x over previous
import functools

import numpy as np
import jax
import jax.numpy as jnp
from jax import lax
from jax.experimental import pallas as pl
from jax.experimental.pallas import tpu as pltpu
from jax.experimental.pallas import tpu_sc as plsc

_N, _D, _H, _E, _K = 2048, 768, 3072, 8, 2
_BLOCK = 256
_NBLK = (_N * _K) // _BLOCK + _E
_PAD = _NBLK * _BLOCK
_NPAIR = _N * _K
_EPS = float(np.finfo(np.float64).eps)
_NW = 32


def _gate_body(x_ref, wg_ref, p_ref, g_ref, ri_ref, first_ref, rex_ref,
               nr_ref, bu_ref, xsm_ref):
    x = x_ref[...]
    wg = wg_ref[...]
    logits = lax.dot_general(x, wg, (((1,), (1,)), ((), ())),
                             preferred_element_type=jnp.float32)
    ioe = lax.broadcasted_iota(jnp.int32, (_N, _E), 1)
    m1 = jnp.max(logits, axis=1, keepdims=True)
    i1 = jnp.min(jnp.where(logits == m1, ioe, _E), axis=1, keepdims=True)
    l2 = jnp.where(ioe == i1, jnp.float32(-jnp.inf), logits)
    m2 = jnp.max(l2, axis=1, keepdims=True)
    i2 = jnp.min(jnp.where(l2 == m2, ioe, _E), axis=1, keepdims=True)
    e21 = jnp.exp(m2 - m1)
    g1 = 1.0 / (1.0 + e21)
    g2 = e21 / (1.0 + e21)

    oh1 = (ioe == i1).astype(jnp.float32)
    oh2 = (ioe == i2).astype(jnp.float32)

    bs = 512
    nchunk = _NPAIR // bs
    ti = lax.broadcasted_iota(jnp.int32, (bs, bs), 0)
    tj = lax.broadcasted_iota(jnp.int32, (bs, bs), 1)
    tri = (ti > tj).astype(jnp.float32)
    run = jnp.zeros((1, _E), jnp.float32)
    rank_chunks = []
    for j in range(nchunk):
        if j < nchunk // 2:
            mj = oh1[j * bs:(j + 1) * bs]
        else:
            mj = oh2[(j - nchunk // 2) * bs:(j - nchunk // 2 + 1) * bs]
        rank_chunks.append(
            lax.dot_general(tri, mj, (((1,), (0,)), ((), ())),
                            preferred_element_type=jnp.float32) + run)
        run = run + jnp.sum(mj, axis=0, keepdims=True)
    counts = run

    nb = jnp.floor((counts + (_BLOCK - 1)) / _BLOCK)
    si = lax.broadcasted_iota(jnp.int32, (_E, _E), 0)
    sj = lax.broadcasted_iota(jnp.int32, (_E, _E), 1)
    sl = (si < sj).astype(jnp.float32)
    bo = lax.dot_general(nb, sl, (((1,), (0,)), ((), ())),
                         preferred_element_type=jnp.float32)
    cnb = bo + nb

    p_chunks = []
    for j in range(nchunk):
        if j < nchunk // 2:
            mj = oh1[j * bs:(j + 1) * bs]
        else:
            mj = oh2[(j - nchunk // 2) * bs:(j - nchunk // 2 + 1) * bs]
        sb = lax.dot_general(mj, bo * _BLOCK, (((1,), (1,)), ((), ())),
                             preferred_element_type=jnp.float32)
        rr = jnp.sum(rank_chunks[j] * mj, axis=1, keepdims=True)
        p_chunks.append(sb + rr)
    p_ref[...] = jnp.concatenate(p_chunks, axis=0).astype(jnp.int32)
    g_ref[...] = jnp.concatenate([g1, g2], axis=0)

    iobi = lax.broadcasted_iota(jnp.int32, (_NBLK, _E), 0)
    nbpos = nb > 0
    first_m = jnp.logical_and(iobi == bo.astype(jnp.int32),
                              jnp.broadcast_to(nbpos, (_NBLK, _E)))
    first = jnp.sum(first_m.astype(jnp.float32), axis=1, keepdims=True)
    t24i = lax.broadcasted_iota(jnp.int32, (_NBLK, _NBLK), 0)
    t24j = lax.broadcasted_iota(jnp.int32, (_NBLK, _NBLK), 1)
    t24 = (t24i >= t24j).astype(jnp.float32)
    ri = lax.dot_general(t24, first, (((1,), (0,)), ((), ())),
                         preferred_element_type=jnp.float32) - 1.0
    ridx = lax.dot_general(nbpos.astype(jnp.float32), sl,
                           (((1,), (0,)), ((), ())),
                           preferred_element_type=jnp.float32)
    iorr = lax.broadcasted_iota(jnp.int32, (_E, _E), 0)
    run_m = jnp.logical_and(iorr == ridx.astype(jnp.int32),
                            jnp.broadcast_to(nbpos, (_E, _E)))
    lanecol = lax.broadcasted_iota(jnp.int32, (_E, 1), 0).astype(jnp.float32)
    rex = lax.dot_general(run_m.astype(jnp.float32), lanecol,
                          (((1,), (0,)), ((), ())),
                          preferred_element_type=jnp.float32)
    nr = jnp.sum(nbpos.astype(jnp.float32), axis=1, keepdims=True)
    ri_ref[...] = ri.astype(jnp.int32)
    first_ref[...] = first.astype(jnp.int32)
    rex_ref[...] = rex.astype(jnp.int32)
    nr_ref[...] = nr.astype(jnp.int32)
    total = jnp.sum(nb, axis=1, keepdims=True)
    iob1 = lax.broadcasted_iota(jnp.int32, (_NBLK, 1), 0).astype(jnp.float32)
    bu_ref[...] = (iob1 < total).astype(jnp.int32)
    xsm_ref[...] = jnp.minimum(iob1, total - 1.0).astype(jnp.int32)


def _gating(x, w_gate):
    return pl.pallas_call(
        _gate_body,
        out_shape=(
            jax.ShapeDtypeStruct((_NPAIR, 1), jnp.int32),
            jax.ShapeDtypeStruct((_NPAIR, 1), jnp.float32),
            jax.ShapeDtypeStruct((_NBLK, 1), jnp.int32),
            jax.ShapeDtypeStruct((_NBLK, 1), jnp.int32),
            jax.ShapeDtypeStruct((_E, 1), jnp.int32),
            jax.ShapeDtypeStruct((1, 1), jnp.int32),
            jax.ShapeDtypeStruct((_NBLK, 1), jnp.int32),
            jax.ShapeDtypeStruct((_NBLK, 1), jnp.int32),
        ),
    )(x, w_gate)


def _dispatch(x, p):
    ch = _NPAIR // _NW
    hc = ch // 2
    mesh = plsc.VectorSubcoreMesh(core_axis_name="c", subcore_axis_name="s")

    @functools.partial(
        pl.kernel, mesh=mesh,
        out_type=jax.ShapeDtypeStruct((_PAD, _D), jnp.float32),
        scratch_types=[
            pltpu.VMEM((2, hc), jnp.int32),
            pltpu.VMEM((2, hc, _D), jnp.float32),
            pltpu.SemaphoreType.DMA((2,)),
            pltpu.SemaphoreType.DMA((2,)),
        ],
    )
    def k(x_hbm, p_hbm, xs_hbm, idx_v, rows_v, lsem, ssem):
        wid = lax.axis_index("s") * 2 + lax.axis_index("c")
        base = wid * ch
        xb = lax.rem(base, _N)
        pltpu.sync_copy(p_hbm.at[pl.ds(base, hc)], idx_v.at[0])
        pltpu.sync_copy(p_hbm.at[pl.ds(base + hc, hc)], idx_v.at[1])
        l0 = pltpu.make_async_copy(x_hbm.at[pl.ds(xb, hc)],
                                   rows_v.at[0], lsem.at[0])
        l1 = pltpu.make_async_copy(x_hbm.at[pl.ds(xb + hc, hc)],
                                   rows_v.at[1], lsem.at[1])
        l0.start()
        l1.start()
        l0.wait()
        s0 = pltpu.make_async_copy(rows_v.at[0], xs_hbm.at[idx_v.at[0]],
                                   ssem.at[0])
        s0.start()
        l1.wait()
        s1 = pltpu.make_async_copy(rows_v.at[1], xs_hbm.at[idx_v.at[1]],
                                   ssem.at[1])
        s1.start()
        s0.wait()
        s1.wait()

    return k(x, p)


def _ffn_body(ri_ref, first_ref, rex_ref, nr_ref, bu_ref, xsm_ref,
              xs_ref, w1_hbm, b1_ref, w2_hbm, b2_ref, out_ref,
              w1_buf, w2_buf, sem1, sem2):
    b = pl.program_id(0)
    ri = ri_ref[b]
    slot = lax.rem(ri, 2)

    def _start_fetch(run, slot_):
        e = rex_ref[run]
        pltpu.make_async_copy(w1_hbm.at[e], w1_buf.at[slot_],
                              sem1.at[slot_]).start()
        pltpu.make_async_copy(w2_hbm.at[e], w2_buf.at[slot_],
                              sem2.at[slot_]).start()

    @pl.when(b == 0)
    def _():
        _start_fetch(0, 0)

    @pl.when(first_ref[b] == 1)
    def _():
        nxt = ri + 1

        @pl.when(nxt < nr_ref[0])
        def _():
            _start_fetch(nxt, lax.rem(nxt, 2))

        e = rex_ref[ri]
        pltpu.make_async_copy(w1_hbm.at[e], w1_buf.at[slot],
                              sem1.at[slot]).wait()
        pltpu.make_async_copy(w2_hbm.at[e], w2_buf.at[slot],
                              sem2.at[slot]).wait()

    @pl.when(bu_ref[b] == 1)
    def _():
        e = rex_ref[ri]
        xs = xs_ref[...].astype(jnp.bfloat16)
        hh = _H // 2
        o = jnp.zeros((_BLOCK, _D), jnp.float32) + b2_ref[e][None, :]
        for j in range(2):
            w1h = w1_buf[slot, pl.ds(j * hh, hh), :].astype(jnp.bfloat16)
            hv = lax.dot_general(xs, w1h, (((1,), (1,)), ((), ())),
                                 preferred_element_type=jnp.float32)
            hv = jnp.maximum(hv + b1_ref[e, pl.ds(j * hh, hh)][None, :], 0.0)
            w2h = w2_buf[slot, :, pl.ds(j * hh, hh)].astype(jnp.bfloat16)
            o = o + lax.dot_general(hv.astype(jnp.bfloat16), w2h,
                                    (((1,), (1,)), ((), ())),
                                    preferred_element_type=jnp.float32)
        mx = jnp.max(o, axis=1, keepdims=True)
        ex = jnp.exp(o - mx)
        out_ref[...] = ex / jnp.sum(ex, axis=1, keepdims=True)


def _ffn(xs, W1, b1, W2, b2, ri, first, rex, nr, bu, xsm):
    grid_spec = pltpu.PrefetchScalarGridSpec(
        num_scalar_prefetch=6,
        grid=(_NBLK,),
        in_specs=[
            pl.BlockSpec((_BLOCK, _D),
                         lambda b, ri, fi, re, nr, bu, xsm: (xsm[b], 0)),
            pl.BlockSpec(memory_space=pl.ANY),
            pl.BlockSpec((_E, _H), lambda b, *_: (0, 0)),
            pl.BlockSpec(memory_space=pl.ANY),
            pl.BlockSpec((_E, _D), lambda b, *_: (0, 0)),
        ],
        out_specs=pl.BlockSpec((_BLOCK, _D),
                               lambda b, ri, fi, re, nr, bu, xsm: (xsm[b], 0)),
        scratch_shapes=[
            pltpu.VMEM((2, _H, _D), jnp.float32),
            pltpu.VMEM((2, _D, _H), jnp.float32),
            pltpu.SemaphoreType.DMA((2,)),
            pltpu.SemaphoreType.DMA((2,)),
        ],
    )
    return pl.pallas_call(
        _ffn_body,
        grid_spec=grid_spec,
        out_shape=jax.ShapeDtypeStruct((_PAD, _D), jnp.float32),
        compiler_params=pltpu.CompilerParams(
            vmem_limit_bytes=112 * 1024 * 1024),
    )(ri, first, rex, nr, bu, xsm, xs, W1, b1, W2, b2)


def _combine_gather(contrib, p):
    ch = _N // _NW
    mesh = plsc.VectorSubcoreMesh(core_axis_name="c", subcore_axis_name="s")

    @functools.partial(
        pl.kernel, mesh=mesh,
        out_type=(
            jax.ShapeDtypeStruct((_N, _D), jnp.float32),
            jax.ShapeDtypeStruct((_N, _D), jnp.float32),
        ),
        scratch_types=[
            pltpu.VMEM((ch,), jnp.int32),
            pltpu.VMEM((ch,), jnp.int32),
            pltpu.VMEM((ch, _D), jnp.float32),
            pltpu.VMEM((ch, _D), jnp.float32),
            pltpu.SemaphoreType.DMA,
            pltpu.SemaphoreType.DMA,
        ],
    )
    def k(contrib_hbm, p_hbm, c1_hbm, c2_hbm, i1v, i2v, r1v, r2v, s1, s2):
        wid = lax.axis_index("s") * 2 + lax.axis_index("c")
        base = wid * ch
        pltpu.sync_copy(p_hbm.at[pl.ds(base, ch)], i1v)
        pltpu.sync_copy(p_hbm.at[pl.ds(_N + base, ch)], i2v)
        cp1 = pltpu.async_copy(contrib_hbm.at[i1v], r1v, s1)
        cp2 = pltpu.async_copy(contrib_hbm.at[i2v], r2v, s2)
        cp1.wait()
        w1 = pltpu.make_async_copy(r1v, c1_hbm.at[pl.ds(base, ch)], s1)
        w1.start()
        cp2.wait()
        w2 = pltpu.make_async_copy(r2v, c2_hbm.at[pl.ds(base, ch)], s2)
        w2.start()
        w1.wait()
        w2.wait()

    return k(contrib, p)


def _combine_body(c1_ref, c2_ref, g1_ref, g2_ref, out_ref):
    c = g1_ref[...] * c1_ref[...] + g2_ref[...] * c2_ref[...]
    c = jnp.where(c == 0.0, jnp.float32(_EPS), c)
    out_ref[...] = jnp.log(c)


def _combine(c1, c2, g):
    nb = _N // _BLOCK
    return pl.pallas_call(
        _combine_body,
        grid=(nb,),
        in_specs=[
            pl.BlockSpec((_BLOCK, _D), lambda i: (i, 0)),
            pl.BlockSpec((_BLOCK, _D), lambda i: (i, 0)),
            pl.BlockSpec((_BLOCK, 1), lambda i: (i, 0)),
            pl.BlockSpec((_BLOCK, 1), lambda i: (nb + i, 0)),
        ],
        out_specs=pl.BlockSpec((_BLOCK, _D), lambda i: (i, 0)),
        out_shape=jax.ShapeDtypeStruct((_N, _D), jnp.float32),
    )(c1, c2, g, g)


def kernel(x, w_gate, W1, b1, W2, b2):
    p2, g2_, ri2, first2, rex2, nr2, bu2, xsm2 = _gating(x, w_gate)
    p = p2.reshape((_NPAIR,))
    ri = ri2.reshape((_NBLK,))
    first = first2.reshape((_NBLK,))
    rex = rex2.reshape((_E,))
    nr = nr2.reshape((1,))
    bu = bu2.reshape((_NBLK,))
    xsm = xsm2.reshape((_NBLK,))
    xs = _dispatch(x, p)
    contrib = _ffn(xs, W1, b1, W2, b2, ri, first, rex, nr, bu, xsm)
    c1, c2 = _combine_gather(contrib, p)
    return _combine(c1, c2, g2_)


# speedup vs baseline: 1.4698x; 1.0079x over previous
import functools

import numpy as np
import jax
import jax.numpy as jnp
from jax import lax
from jax.experimental import pallas as pl
from jax.experimental.pallas import tpu as pltpu
from jax.experimental.pallas import tpu_sc as plsc

_N, _D, _H, _E, _K = 2048, 768, 3072, 8, 2
_BLOCK = 256
_NBLK = (_N * _K) // _BLOCK + _E
_PAD = _NBLK * _BLOCK
_NPAIR = _N * _K
_EPS = float(np.finfo(np.float64).eps)
_NW = 32


def _gate_body(x_ref, wg_ref, p_ref, g_ref, ri_ref, first_ref, rex_ref,
               nr_ref, bu_ref, xsm_ref):
    x = x_ref[...]
    wg = wg_ref[...]
    logits = lax.dot_general(x, wg, (((1,), (1,)), ((), ())),
                             preferred_element_type=jnp.float32)
    ioe = lax.broadcasted_iota(jnp.int32, (_N, _E), 1)
    m1 = jnp.max(logits, axis=1, keepdims=True)
    i1 = jnp.min(jnp.where(logits == m1, ioe, _E), axis=1, keepdims=True)
    l2 = jnp.where(ioe == i1, jnp.float32(-jnp.inf), logits)
    m2 = jnp.max(l2, axis=1, keepdims=True)
    i2 = jnp.min(jnp.where(l2 == m2, ioe, _E), axis=1, keepdims=True)
    e21 = jnp.exp(m2 - m1)
    g1 = 1.0 / (1.0 + e21)
    g2 = e21 / (1.0 + e21)

    oh1 = (ioe == i1).astype(jnp.float32)
    oh2 = (ioe == i2).astype(jnp.float32)

    bs = 512
    nchunk = _NPAIR // bs
    ti = lax.broadcasted_iota(jnp.int32, (bs, bs), 0)
    tj = lax.broadcasted_iota(jnp.int32, (bs, bs), 1)
    tri = (ti > tj).astype(jnp.float32)
    run = jnp.zeros((1, _E), jnp.float32)
    rank_chunks = []
    for j in range(nchunk):
        if j < nchunk // 2:
            mj = oh1[j * bs:(j + 1) * bs]
        else:
            mj = oh2[(j - nchunk // 2) * bs:(j - nchunk // 2 + 1) * bs]
        rank_chunks.append(
            lax.dot_general(tri, mj, (((1,), (0,)), ((), ())),
                            preferred_element_type=jnp.float32) + run)
        run = run + jnp.sum(mj, axis=0, keepdims=True)
    counts = run

    nb = jnp.floor((counts + (_BLOCK - 1)) / _BLOCK)
    si = lax.broadcasted_iota(jnp.int32, (_E, _E), 0)
    sj = lax.broadcasted_iota(jnp.int32, (_E, _E), 1)
    sl = (si < sj).astype(jnp.float32)
    bo = lax.dot_general(nb, sl, (((1,), (0,)), ((), ())),
                         preferred_element_type=jnp.float32)
    cnb = bo + nb

    p_chunks = []
    for j in range(nchunk):
        if j < nchunk // 2:
            mj = oh1[j * bs:(j + 1) * bs]
        else:
            mj = oh2[(j - nchunk // 2) * bs:(j - nchunk // 2 + 1) * bs]
        sb = lax.dot_general(mj, bo * _BLOCK, (((1,), (1,)), ((), ())),
                             preferred_element_type=jnp.float32)
        rr = jnp.sum(rank_chunks[j] * mj, axis=1, keepdims=True)
        p_chunks.append(sb + rr)
    p_ref[...] = jnp.concatenate(p_chunks, axis=0).astype(jnp.int32)
    g_ref[...] = jnp.concatenate([g1, g2], axis=0)

    iobi = lax.broadcasted_iota(jnp.int32, (_NBLK, _E), 0)
    nbpos = nb > 0
    first_m = jnp.logical_and(iobi == bo.astype(jnp.int32),
                              jnp.broadcast_to(nbpos, (_NBLK, _E)))
    first = jnp.sum(first_m.astype(jnp.float32), axis=1, keepdims=True)
    t24i = lax.broadcasted_iota(jnp.int32, (_NBLK, _NBLK), 0)
    t24j = lax.broadcasted_iota(jnp.int32, (_NBLK, _NBLK), 1)
    t24 = (t24i >= t24j).astype(jnp.float32)
    ri = lax.dot_general(t24, first, (((1,), (0,)), ((), ())),
                         preferred_element_type=jnp.float32) - 1.0
    ridx = lax.dot_general(nbpos.astype(jnp.float32), sl,
                           (((1,), (0,)), ((), ())),
                           preferred_element_type=jnp.float32)
    iorr = lax.broadcasted_iota(jnp.int32, (_E, _E), 0)
    run_m = jnp.logical_and(iorr == ridx.astype(jnp.int32),
                            jnp.broadcast_to(nbpos, (_E, _E)))
    lanecol = lax.broadcasted_iota(jnp.int32, (_E, 1), 0).astype(jnp.float32)
    rex = lax.dot_general(run_m.astype(jnp.float32), lanecol,
                          (((1,), (0,)), ((), ())),
                          preferred_element_type=jnp.float32)
    nr = jnp.sum(nbpos.astype(jnp.float32), axis=1, keepdims=True)
    ri_ref[...] = ri.astype(jnp.int32)
    first_ref[...] = first.astype(jnp.int32)
    rex_ref[...] = rex.astype(jnp.int32)
    nr_ref[...] = nr.astype(jnp.int32)
    total = jnp.sum(nb, axis=1, keepdims=True)
    iob1 = lax.broadcasted_iota(jnp.int32, (_NBLK, 1), 0).astype(jnp.float32)
    bu_ref[...] = (iob1 < total).astype(jnp.int32)
    xsm_ref[...] = jnp.minimum(iob1, total - 1.0).astype(jnp.int32)


def _gating(x, w_gate):
    return pl.pallas_call(
        _gate_body,
        out_shape=(
            jax.ShapeDtypeStruct((_NPAIR, 1), jnp.int32),
            jax.ShapeDtypeStruct((_NPAIR, 1), jnp.float32),
            jax.ShapeDtypeStruct((_NBLK, 1), jnp.int32),
            jax.ShapeDtypeStruct((_NBLK, 1), jnp.int32),
            jax.ShapeDtypeStruct((_E, 1), jnp.int32),
            jax.ShapeDtypeStruct((1, 1), jnp.int32),
            jax.ShapeDtypeStruct((_NBLK, 1), jnp.int32),
            jax.ShapeDtypeStruct((_NBLK, 1), jnp.int32),
        ),
    )(x, w_gate)


def _dispatch(x, p):
    ch = _NPAIR // _NW
    hc = ch // 2
    mesh = plsc.VectorSubcoreMesh(core_axis_name="c", subcore_axis_name="s")

    @functools.partial(
        pl.kernel, mesh=mesh,
        out_type=jax.ShapeDtypeStruct((_PAD, _D), jnp.float32),
        scratch_types=[
            pltpu.VMEM((2, hc), jnp.int32),
            pltpu.VMEM((2, hc, _D), jnp.float32),
            pltpu.SemaphoreType.DMA((2,)),
            pltpu.SemaphoreType.DMA((2,)),
        ],
    )
    def k(x_hbm, p_hbm, xs_hbm, idx_v, rows_v, lsem, ssem):
        wid = lax.axis_index("s") * 2 + lax.axis_index("c")
        base = wid * ch
        xb = lax.rem(base, _N)
        l0 = pltpu.make_async_copy(x_hbm.at[pl.ds(xb, hc)],
                                   rows_v.at[0], lsem.at[0])
        l1 = pltpu.make_async_copy(x_hbm.at[pl.ds(xb + hc, hc)],
                                   rows_v.at[1], lsem.at[1])
        l0.start()
        l1.start()
        pltpu.sync_copy(p_hbm.at[pl.ds(base, hc)], idx_v.at[0])
        pltpu.sync_copy(p_hbm.at[pl.ds(base + hc, hc)], idx_v.at[1])
        l0.wait()
        s0 = pltpu.make_async_copy(rows_v.at[0], xs_hbm.at[idx_v.at[0]],
                                   ssem.at[0])
        s0.start()
        l1.wait()
        s1 = pltpu.make_async_copy(rows_v.at[1], xs_hbm.at[idx_v.at[1]],
                                   ssem.at[1])
        s1.start()
        s0.wait()
        s1.wait()

    return k(x, p)


def _ffn_body(ri_ref, first_ref, rex_ref, nr_ref, bu_ref, xsm_ref,
              xs_ref, w1_hbm, b1_ref, w2_hbm, b2_ref, out_ref,
              w1_buf, w2_buf, sem1, sem2):
    b = pl.program_id(0)
    ri = ri_ref[b]
    slot = lax.rem(ri, 2)

    def _start_fetch(run, slot_):
        e = rex_ref[run]
        pltpu.make_async_copy(w1_hbm.at[e], w1_buf.at[slot_],
                              sem1.at[slot_]).start()
        pltpu.make_async_copy(w2_hbm.at[e], w2_buf.at[slot_],
                              sem2.at[slot_]).start()

    @pl.when(b == 0)
    def _():
        _start_fetch(0, 0)

    @pl.when(first_ref[b] == 1)
    def _():
        nxt = ri + 1

        @pl.when(nxt < nr_ref[0])
        def _():
            _start_fetch(nxt, lax.rem(nxt, 2))

        e = rex_ref[ri]
        pltpu.make_async_copy(w1_hbm.at[e], w1_buf.at[slot],
                              sem1.at[slot]).wait()
        pltpu.make_async_copy(w2_hbm.at[e], w2_buf.at[slot],
                              sem2.at[slot]).wait()

    @pl.when(bu_ref[b] == 1)
    def _():
        e = rex_ref[ri]
        xs = xs_ref[...].astype(jnp.bfloat16)
        hh = _H // 2
        o = jnp.zeros((_BLOCK, _D), jnp.float32) + b2_ref[e][None, :]
        for j in range(2):
            w1h = w1_buf[slot, pl.ds(j * hh, hh), :].astype(jnp.bfloat16)
            hv = lax.dot_general(xs, w1h, (((1,), (1,)), ((), ())),
                                 preferred_element_type=jnp.float32)
            hv = jnp.maximum(hv + b1_ref[e, pl.ds(j * hh, hh)][None, :], 0.0)
            w2h = w2_buf[slot, :, pl.ds(j * hh, hh)].astype(jnp.bfloat16)
            o = o + lax.dot_general(hv.astype(jnp.bfloat16), w2h,
                                    (((1,), (1,)), ((), ())),
                                    preferred_element_type=jnp.float32)
        mx = jnp.max(o, axis=1, keepdims=True)
        ex = jnp.exp(o - mx)
        out_ref[...] = ex / jnp.sum(ex, axis=1, keepdims=True)


def _ffn(xs, W1, b1, W2, b2, ri, first, rex, nr, bu, xsm):
    grid_spec = pltpu.PrefetchScalarGridSpec(
        num_scalar_prefetch=6,
        grid=(_NBLK,),
        in_specs=[
            pl.BlockSpec((_BLOCK, _D),
                         lambda b, ri, fi, re, nr, bu, xsm: (xsm[b], 0)),
            pl.BlockSpec(memory_space=pl.ANY),
            pl.BlockSpec((_E, _H), lambda b, *_: (0, 0)),
            pl.BlockSpec(memory_space=pl.ANY),
            pl.BlockSpec((_E, _D), lambda b, *_: (0, 0)),
        ],
        out_specs=pl.BlockSpec((_BLOCK, _D),
                               lambda b, ri, fi, re, nr, bu, xsm: (xsm[b], 0)),
        scratch_shapes=[
            pltpu.VMEM((2, _H, _D), jnp.float32),
            pltpu.VMEM((2, _D, _H), jnp.float32),
            pltpu.SemaphoreType.DMA((2,)),
            pltpu.SemaphoreType.DMA((2,)),
        ],
    )
    return pl.pallas_call(
        _ffn_body,
        grid_spec=grid_spec,
        out_shape=jax.ShapeDtypeStruct((_PAD, _D), jnp.float32),
        compiler_params=pltpu.CompilerParams(
            vmem_limit_bytes=112 * 1024 * 1024),
    )(ri, first, rex, nr, bu, xsm, xs, W1, b1, W2, b2)


def _combine_gather(contrib, p):
    ch = _N // _NW
    mesh = plsc.VectorSubcoreMesh(core_axis_name="c", subcore_axis_name="s")

    @functools.partial(
        pl.kernel, mesh=mesh,
        out_type=(
            jax.ShapeDtypeStruct((_N, _D), jnp.float32),
            jax.ShapeDtypeStruct((_N, _D), jnp.float32),
        ),
        scratch_types=[
            pltpu.VMEM((ch,), jnp.int32),
            pltpu.VMEM((ch,), jnp.int32),
            pltpu.VMEM((ch, _D), jnp.float32),
            pltpu.VMEM((ch, _D), jnp.float32),
            pltpu.SemaphoreType.DMA,
            pltpu.SemaphoreType.DMA,
        ],
    )
    def k(contrib_hbm, p_hbm, c1_hbm, c2_hbm, i1v, i2v, r1v, r2v, s1, s2):
        wid = lax.axis_index("s") * 2 + lax.axis_index("c")
        base = wid * ch
        a1 = pltpu.make_async_copy(p_hbm.at[pl.ds(base, ch)], i1v, s1)
        a2 = pltpu.make_async_copy(p_hbm.at[pl.ds(_N + base, ch)], i2v, s2)
        a1.start()
        a2.start()
        a1.wait()
        cp1 = pltpu.async_copy(contrib_hbm.at[i1v], r1v, s1)
        a2.wait()
        cp2 = pltpu.async_copy(contrib_hbm.at[i2v], r2v, s2)
        cp1.wait()
        w1 = pltpu.make_async_copy(r1v, c1_hbm.at[pl.ds(base, ch)], s1)
        w1.start()
        cp2.wait()
        w2 = pltpu.make_async_copy(r2v, c2_hbm.at[pl.ds(base, ch)], s2)
        w2.start()
        w1.wait()
        w2.wait()

    return k(contrib, p)


def _combine_body(c1_ref, c2_ref, g1_ref, g2_ref, out_ref):
    c = g1_ref[...] * c1_ref[...] + g2_ref[...] * c2_ref[...]
    c = jnp.where(c == 0.0, jnp.float32(_EPS), c)
    out_ref[...] = jnp.log(c)


def _combine(c1, c2, g):
    nb = _N // _BLOCK
    return pl.pallas_call(
        _combine_body,
        grid=(nb,),
        in_specs=[
            pl.BlockSpec((_BLOCK, _D), lambda i: (i, 0)),
            pl.BlockSpec((_BLOCK, _D), lambda i: (i, 0)),
            pl.BlockSpec((_BLOCK, 1), lambda i: (i, 0)),
            pl.BlockSpec((_BLOCK, 1), lambda i: (nb + i, 0)),
        ],
        out_specs=pl.BlockSpec((_BLOCK, _D), lambda i: (i, 0)),
        out_shape=jax.ShapeDtypeStruct((_N, _D), jnp.float32),
    )(c1, c2, g, g)


def kernel(x, w_gate, W1, b1, W2, b2):
    p2, g2_, ri2, first2, rex2, nr2, bu2, xsm2 = _gating(x, w_gate)
    p = p2.reshape((_NPAIR,))
    ri = ri2.reshape((_NBLK,))
    first = first2.reshape((_NBLK,))
    rex = rex2.reshape((_E,))
    nr = nr2.reshape((1,))
    bu = bu2.reshape((_NBLK,))
    xsm = xsm2.reshape((_NBLK,))
    xs = _dispatch(x, p)
    contrib = _ffn(xs, W1, b1, W2, b2, ri, first, rex, nr, bu, xsm)
    c1, c2 = _combine_gather(contrib, p)
    return _combine(c1, c2, g2_)
